# baseline (speedup 1.0000x reference)
import functools
import math

import jax
import jax.numpy as jnp
import numpy as np
from jax import lax
from jax.experimental import pallas as pl
from jax.experimental.pallas import tpu as pltpu
from jax.experimental.pallas import tpu_sc as plsc

N = 10000
E = 320000
D = 128
K = 10

NC = 2
NS = 16
NW = NC * NS
EPW = E // NW
CH = 80
NCHUNK = 128
EPAD = NCHUNK * CH - EPW
NACC = 10240
RPS = NACC // NS

_TCR = 1000
_GRID = N // _TCR


def _seg_body(v_hbm, cr_hbm, zero_hbm, p_hbm, colv0, colv1, colv2, colv3,
              rowv0, rowv1, rowv2, rowv3, gbuf, gbuf1, gbuf2, gbuf3, acc,
              gsem0, gsem1, gsem2, gsem3, ssem0, ssem1, ssem2, ssem3):
    c = lax.axis_index("c")
    s = lax.axis_index("s")
    wid = c * NS + s
    colv = (colv0, colv1, colv2, colv3)
    rowv = (rowv0, rowv1, rowv2, rowv3)
    gbufs = (gbuf, gbuf1, gbuf2, gbuf3)
    gsems = (gsem0, gsem1, gsem2, gsem3)
    ssems = (ssem0, ssem1, ssem2, ssem3)

    pltpu.sync_copy(zero_hbm, gbuf)
    rbase = pl.multiple_of(s * RPS, 8)
    for h in range(RPS // CH):
        pltpu.sync_copy(gbuf, acc.at[pl.ds(rbase + h * CH, CH)])
    plsc.subcore_barrier()

    pltpu.sync_copy(cr_hbm.at[wid, 0, 0], colv0)
    pltpu.sync_copy(cr_hbm.at[wid, 0, 1], rowv0)
    pltpu.async_copy(v_hbm.at[colv0], gbuf, gsem0)

    def trip(p, carry):
        for i in range(4):
            j = p * 4 + i
            nb = (i + 1) % 4

            @pl.when(j + 1 < NCHUNK)
            def _():
                @pl.when(j >= 3)
                def _():
                    pltpu.make_async_copy(zero_hbm, gbufs[nb],
                                          ssems[nb]).wait()
                pltpu.sync_copy(cr_hbm.at[wid, j + 1, 0], colv[nb])
                pltpu.sync_copy(cr_hbm.at[wid, j + 1, 1], rowv[nb])
                pltpu.async_copy(v_hbm.at[colv[nb]], gbufs[nb], gsems[nb])

            pltpu.make_async_copy(zero_hbm, gbufs[i], gsems[i]).wait()
            pltpu.async_copy(gbufs[i], acc.at[rowv[i]], ssems[i], add=True)
        return carry

    lax.fori_loop(0, NCHUNK // 4, trip, 0)
    for b in range(4):
        pltpu.make_async_copy(zero_hbm, gbufs[b], ssems[b]).wait()
    plsc.subcore_barrier()

    nh = jnp.where(s == NS - 1, (N - (NS - 1) * RPS) // CH, RPS // CH)

    def ohop(h, carry):
        rb = pl.multiple_of(rbase + h * CH, 8)
        pltpu.sync_copy(acc.at[pl.ds(rb, CH)], gbuf)
        pltpu.sync_copy(gbuf, p_hbm.at[pl.ds(c * N + rb, CH)])
        return carry

    lax.fori_loop(0, nh, ohop, 0)

    _TB = (NS - 1) * RPS + ((N - (NS - 1) * RPS) // CH) * CH
    if N > _TB:
        @pl.when(s == NS - 1)
        def _():
            pltpu.sync_copy(acc.at[pl.ds(_TB, N - _TB)],
                            gbuf.at[pl.ds(0, N - _TB)])
            pltpu.sync_copy(gbuf.at[pl.ds(0, N - _TB)],
                            p_hbm.at[pl.ds(c * N + _TB, N - _TB)])


_seg = pl.kernel(
    _seg_body,
    out_type=jax.ShapeDtypeStruct((NC * N, D), jnp.float32),
    mesh=plsc.VectorSubcoreMesh(core_axis_name="c", subcore_axis_name="s"),
    scratch_types=(
        [pltpu.VMEM((CH,), jnp.int32)] * 4
        + [pltpu.VMEM((CH,), jnp.int32)] * 4
        + [pltpu.VMEM((CH, D), jnp.float32)] * 4
        + [pltpu.VMEM_SHARED((NACC, D), jnp.float32)]
        + [pltpu.SemaphoreType.DMA] * 8
    ),
)


def _prep_body(c0_ref, p_ref, x_ref, dinv_ref, g_ref, out_ref):
    s = p_ref[0] + p_ref[1]
    dinv = jnp.where(s > 0, lax.rsqrt(jnp.maximum(s, 1e-12)), 0.0)
    x = x_ref[...]
    dinv_ref[...] = dinv
    g_ref[...] = dinv * x
    out_ref[...] = c0_ref[0, 0] * x


_prep = pl.pallas_call(
    _prep_body,
    grid=(_GRID,),
    in_specs=[
        pl.BlockSpec(memory_space=pltpu.SMEM),
        pl.BlockSpec((2, _TCR, D), lambda i: (0, i, 0)),
        pl.BlockSpec((_TCR, D), lambda i: (i, 0)),
    ],
    out_specs=[
        pl.BlockSpec((_TCR, D), lambda i: (i, 0)),
        pl.BlockSpec((_TCR, D), lambda i: (i, 0)),
        pl.BlockSpec((_TCR, D), lambda i: (i, 0)),
    ],
    out_shape=[jax.ShapeDtypeStruct((N, D), jnp.float32)] * 3,
)


def _combine_body(ck_ref, p_ref, tm2_ref, dinv_ref, outin_ref,
                  t_ref, g_ref, outnew_ref, *, first):
    s = p_ref[0] + p_ref[1]
    dinv = dinv_ref[...]
    if first:
        t = -dinv * s
    else:
        t = -2.0 * (dinv * s) - tm2_ref[...]
    t_ref[...] = t
    g_ref[...] = dinv * t
    outnew_ref[...] = outin_ref[...] + ck_ref[0, 0] * t


def _make_combine(first):
    return pl.pallas_call(
        functools.partial(_combine_body, first=first),
        grid=(_GRID,),
        in_specs=[
            pl.BlockSpec(memory_space=pltpu.SMEM),
            pl.BlockSpec((2, _TCR, D), lambda i: (0, i, 0)),
            pl.BlockSpec((_TCR, D), lambda i: (i, 0)),
            pl.BlockSpec((_TCR, D), lambda i: (i, 0)),
            pl.BlockSpec((_TCR, D), lambda i: (i, 0)),
        ],
        out_specs=[
            pl.BlockSpec((_TCR, D), lambda i: (i, 0)),
            pl.BlockSpec((_TCR, D), lambda i: (i, 0)),
            pl.BlockSpec((_TCR, D), lambda i: (i, 0)),
        ],
        out_shape=[jax.ShapeDtypeStruct((N, D), jnp.float32)] * 3,
    )


_combine_first = _make_combine(True)
_combine_rest = _make_combine(False)


def _mid_body(c0_ref, x_ref, xh_ref, td_ref, th1_ref, dinv_ref,
              hid_ref, g_ref, out_ref):
    h = jnp.dot(x_ref[...], td_ref[...], preferred_element_type=jnp.float32)
    h += jnp.dot(xh_ref[...], th1_ref[...], preferred_element_type=jnp.float32)
    h = jnp.maximum(h, 0.0)
    hid_ref[...] = h
    g_ref[...] = dinv_ref[...] * h
    out_ref[...] = c0_ref[0, 0] * h


_mid = pl.pallas_call(
    _mid_body,
    grid=(_GRID,),
    in_specs=[
        pl.BlockSpec(memory_space=pltpu.SMEM),
        pl.BlockSpec((_TCR, D), lambda i: (i, 0)),
        pl.BlockSpec((_TCR, D), lambda i: (i, 0)),
        pl.BlockSpec((D, D), lambda i: (0, 0)),
        pl.BlockSpec((D, D), lambda i: (0, 0)),
        pl.BlockSpec((_TCR, D), lambda i: (i, 0)),
    ],
    out_specs=[
        pl.BlockSpec((_TCR, D), lambda i: (i, 0)),
        pl.BlockSpec((_TCR, D), lambda i: (i, 0)),
        pl.BlockSpec((_TCR, D), lambda i: (i, 0)),
    ],
    out_shape=[jax.ShapeDtypeStruct((N, D), jnp.float32)] * 3,
)


def _final_body(h_ref, hh_ref, th_ref, th2_ref, o_ref):
    z = jnp.dot(h_ref[...], th_ref[...], preferred_element_type=jnp.float32)
    z += jnp.dot(hh_ref[...], th2_ref[...], preferred_element_type=jnp.float32)
    m = jnp.max(z, axis=1, keepdims=True)
    lse = m + jnp.log(jnp.sum(jnp.exp(z - m), axis=1, keepdims=True))
    o_ref[...] = z - lse


_final = pl.pallas_call(
    _final_body,
    grid=(_GRID,),
    in_specs=[
        pl.BlockSpec((_TCR, D), lambda i: (i, 0)),
        pl.BlockSpec((_TCR, D), lambda i: (i, 0)),
        pl.BlockSpec((D, D), lambda i: (0, 0)),
        pl.BlockSpec((D, D), lambda i: (0, 0)),
    ],
    out_specs=pl.BlockSpec((_TCR, D), lambda i: (i, 0)),
    out_shape=jax.ShapeDtypeStruct((N, D), jnp.float32),
)


_M30 = np.arange(30, dtype=np.float32)
_LGAMMA = np.array(
    [[math.lgamma(m + 1.0) + math.lgamma(m + k + 1.0) for m in range(30)]
     for k in range(K)], dtype=np.float32)


def _coeffs(t):
    lt = jnp.log(t / 2.0)
    cs = []
    for k in range(K):
        ik = jnp.sum(jnp.exp((2.0 * _M30 + k) * lt - _LGAMMA[k]))
        ck = ik if k == 0 else 2.0 * ((-1.0) ** k) * ik
        cs.append(jnp.reshape(ck.astype(jnp.float32), (1, 1)))
    return cs


def _heat_sweep(g0, out_acc, x0, cr, zeros, dinv, cs):
    g = g0
    tm2 = x0
    tm1 = None
    for k in range(1, K):
        p = _seg(g, cr, zeros).reshape(NC, N, D)
        comb = _combine_first if k == 1 else _combine_rest
        tk, g, out_acc = comb(cs[k], p, tm2, dinv, out_acc)
        tm2, tm1 = (x0, tk) if k == 1 else (tm1, tk)
    return out_acc


def kernel(x, edge_index, theta_direct, theta_heat1, theta_hidden,
           theta_heat2, t):
    row = edge_index[0]
    col = edge_index[1]
    rowp = jnp.concatenate(
        [row.reshape(NW, EPW),
         jnp.full((NW, EPAD), N, jnp.int32)], axis=1).reshape(NW, NCHUNK, 1,
                                                              CH)
    colp = jnp.concatenate(
        [col.reshape(NW, EPW),
         jnp.zeros((NW, EPAD), jnp.int32)], axis=1).reshape(NW, NCHUNK, 1, CH)
    cr = jnp.concatenate([colp, rowp], axis=2)
    zeros = jnp.zeros((CH, D), jnp.float32)
    ones = jnp.ones((N, D), jnp.float32)
    cs = _coeffs(t)

    pdeg = _seg(ones, cr, zeros).reshape(NC, N, D)
    dinv, g0, out1 = _prep(cs[0], pdeg, x)
    x_heat = _heat_sweep(g0, out1, x, cr, zeros, dinv, cs)

    hidden, gh0, out2 = _mid(cs[0], x, x_heat, theta_direct, theta_heat1,
                             dinv)
    hidden_heat = _heat_sweep(gh0, out2, hidden, cr, zeros, dinv, cs)

    return _final(hidden, hidden_heat, theta_hidden, theta_heat2)


# speedup vs baseline: 1.8389x; 1.8389x over previous
import functools
import math

import jax
import jax.numpy as jnp
import numpy as np
from jax import lax
from jax.experimental import pallas as pl
from jax.experimental.pallas import tpu as pltpu
from jax.experimental.pallas import tpu_sc as plsc

N = 10000
E = 320000
D = 128
K = 10

NC = 2
NS = 16
NW = NC * NS
EPW = E // NW
CH = 96
NCHUNK = 105
EPAD = NCHUNK * CH - EPW
NACC = 10240
RPS = NACC // NS

HOP = 64
_TCR = 1000
_GRID = N // _TCR


def _seg_body(v_hbm, cr_hbm, zero_hbm, p_hbm, colv0, colv1, colv2,
              rowv0, rowv1, rowv2, gbuf, gbuf1, gbuf2, acc,
              gsem0, gsem1, gsem2, ssem0, ssem1, ssem2):
    c = lax.axis_index("c")
    s = lax.axis_index("s")
    wid = c * NS + s
    colv = (colv0, colv1, colv2)
    rowv = (rowv0, rowv1, rowv2)
    gbufs = (gbuf, gbuf1, gbuf2)
    gsems = (gsem0, gsem1, gsem2)
    ssems = (ssem0, ssem1, ssem2)

    pltpu.sync_copy(zero_hbm, gbuf)
    rbase = pl.multiple_of(s * RPS, 8)
    for h in range(RPS // HOP):
        pltpu.sync_copy(gbuf.at[pl.ds(0, HOP)],
                        acc.at[pl.ds(rbase + h * HOP, HOP)])
    plsc.subcore_barrier()

    pltpu.sync_copy(cr_hbm.at[wid, 0, 0], colv0)
    pltpu.sync_copy(cr_hbm.at[wid, 0, 1], rowv0)
    pltpu.async_copy(v_hbm.at[colv0], gbuf, gsem0)

    def trip(p, carry):
        for i in range(3):
            j = p * 3 + i
            nb = (i + 1) % 3

            @pl.when(j + 1 < NCHUNK)
            def _():
                @pl.when(j >= 2)
                def _():
                    pltpu.make_async_copy(zero_hbm, gbufs[nb],
                                          ssems[nb]).wait()
                pltpu.sync_copy(cr_hbm.at[wid, j + 1, 0], colv[nb])
                pltpu.sync_copy(cr_hbm.at[wid, j + 1, 1], rowv[nb])
                pltpu.async_copy(v_hbm.at[colv[nb]], gbufs[nb], gsems[nb])

            pltpu.make_async_copy(zero_hbm, gbufs[i], gsems[i]).wait()
            pltpu.async_copy(gbufs[i], acc.at[rowv[i]], ssems[i], add=True)
        return carry

    lax.fori_loop(0, NCHUNK // 3, trip, 0)
    for b in range(3):
        pltpu.make_async_copy(zero_hbm, gbufs[b], ssems[b]).wait()
    plsc.subcore_barrier()

    nh = jnp.where(s == NS - 1, (N - (NS - 1) * RPS) // HOP, RPS // HOP)

    def ohop(h, carry):
        rb = pl.multiple_of(rbase + h * HOP, 8)
        pltpu.sync_copy(acc.at[pl.ds(rb, HOP)], gbuf.at[pl.ds(0, HOP)])
        pltpu.sync_copy(gbuf.at[pl.ds(0, HOP)],
                        p_hbm.at[pl.ds(c * N + rb, HOP)])
        return carry

    lax.fori_loop(0, nh, ohop, 0)

    _TB = (NS - 1) * RPS + ((N - (NS - 1) * RPS) // HOP) * HOP
    if N > _TB:
        @pl.when(s == NS - 1)
        def _():
            pltpu.sync_copy(acc.at[pl.ds(_TB, N - _TB)],
                            gbuf.at[pl.ds(0, N - _TB)])
            pltpu.sync_copy(gbuf.at[pl.ds(0, N - _TB)],
                            p_hbm.at[pl.ds(c * N + _TB, N - _TB)])


_seg = pl.kernel(
    _seg_body,
    out_type=jax.ShapeDtypeStruct((NC * N, D), jnp.float32),
    mesh=plsc.VectorSubcoreMesh(core_axis_name="c", subcore_axis_name="s"),
    scratch_types=(
        [pltpu.VMEM((CH,), jnp.int32)] * 3
        + [pltpu.VMEM((CH,), jnp.int32)] * 3
        + [pltpu.VMEM((CH, D), jnp.float32)] * 3
        + [pltpu.VMEM_SHARED((NACC, D), jnp.float32)]
        + [pltpu.SemaphoreType.DMA] * 6
    ),
)


def _prep_body(c0_ref, p_ref, x_ref, dinv_ref, g_ref, out_ref):
    s = p_ref[0] + p_ref[1]
    dinv = jnp.where(s > 0, lax.rsqrt(jnp.maximum(s, 1e-12)), 0.0)
    x = x_ref[...]
    dinv_ref[...] = dinv
    g_ref[...] = dinv * x
    out_ref[...] = c0_ref[0, 0] * x


_prep = pl.pallas_call(
    _prep_body,
    grid=(_GRID,),
    in_specs=[
        pl.BlockSpec(memory_space=pltpu.SMEM),
        pl.BlockSpec((2, _TCR, D), lambda i: (0, i, 0)),
        pl.BlockSpec((_TCR, D), lambda i: (i, 0)),
    ],
    out_specs=[
        pl.BlockSpec((_TCR, D), lambda i: (i, 0)),
        pl.BlockSpec((_TCR, D), lambda i: (i, 0)),
        pl.BlockSpec((_TCR, D), lambda i: (i, 0)),
    ],
    out_shape=[jax.ShapeDtypeStruct((N, D), jnp.float32)] * 3,
)


def _combine_body(ck_ref, p_ref, tm2_ref, dinv_ref, outin_ref,
                  t_ref, g_ref, outnew_ref, *, first):
    s = p_ref[0] + p_ref[1]
    dinv = dinv_ref[...]
    if first:
        t = -dinv * s
    else:
        t = -2.0 * (dinv * s) - tm2_ref[...]
    t_ref[...] = t
    g_ref[...] = dinv * t
    outnew_ref[...] = outin_ref[...] + ck_ref[0, 0] * t


def _make_combine(first):
    return pl.pallas_call(
        functools.partial(_combine_body, first=first),
        grid=(_GRID,),
        in_specs=[
            pl.BlockSpec(memory_space=pltpu.SMEM),
            pl.BlockSpec((2, _TCR, D), lambda i: (0, i, 0)),
            pl.BlockSpec((_TCR, D), lambda i: (i, 0)),
            pl.BlockSpec((_TCR, D), lambda i: (i, 0)),
            pl.BlockSpec((_TCR, D), lambda i: (i, 0)),
        ],
        out_specs=[
            pl.BlockSpec((_TCR, D), lambda i: (i, 0)),
            pl.BlockSpec((_TCR, D), lambda i: (i, 0)),
            pl.BlockSpec((_TCR, D), lambda i: (i, 0)),
        ],
        out_shape=[jax.ShapeDtypeStruct((N, D), jnp.float32)] * 3,
    )


_combine_first = _make_combine(True)
_combine_rest = _make_combine(False)


def _mid_body(c0_ref, x_ref, xh_ref, td_ref, th1_ref, dinv_ref,
              hid_ref, g_ref, out_ref):
    h = jnp.dot(x_ref[...], td_ref[...], preferred_element_type=jnp.float32)
    h += jnp.dot(xh_ref[...], th1_ref[...], preferred_element_type=jnp.float32)
    h = jnp.maximum(h, 0.0)
    hid_ref[...] = h
    g_ref[...] = dinv_ref[...] * h
    out_ref[...] = c0_ref[0, 0] * h


_mid = pl.pallas_call(
    _mid_body,
    grid=(_GRID,),
    in_specs=[
        pl.BlockSpec(memory_space=pltpu.SMEM),
        pl.BlockSpec((_TCR, D), lambda i: (i, 0)),
        pl.BlockSpec((_TCR, D), lambda i: (i, 0)),
        pl.BlockSpec((D, D), lambda i: (0, 0)),
        pl.BlockSpec((D, D), lambda i: (0, 0)),
        pl.BlockSpec((_TCR, D), lambda i: (i, 0)),
    ],
    out_specs=[
        pl.BlockSpec((_TCR, D), lambda i: (i, 0)),
        pl.BlockSpec((_TCR, D), lambda i: (i, 0)),
        pl.BlockSpec((_TCR, D), lambda i: (i, 0)),
    ],
    out_shape=[jax.ShapeDtypeStruct((N, D), jnp.float32)] * 3,
)


def _final_body(h_ref, hh_ref, th_ref, th2_ref, o_ref):
    z = jnp.dot(h_ref[...], th_ref[...], preferred_element_type=jnp.float32)
    z += jnp.dot(hh_ref[...], th2_ref[...], preferred_element_type=jnp.float32)
    m = jnp.max(z, axis=1, keepdims=True)
    lse = m + jnp.log(jnp.sum(jnp.exp(z - m), axis=1, keepdims=True))
    o_ref[...] = z - lse


_final = pl.pallas_call(
    _final_body,
    grid=(_GRID,),
    in_specs=[
        pl.BlockSpec((_TCR, D), lambda i: (i, 0)),
        pl.BlockSpec((_TCR, D), lambda i: (i, 0)),
        pl.BlockSpec((D, D), lambda i: (0, 0)),
        pl.BlockSpec((D, D), lambda i: (0, 0)),
    ],
    out_specs=pl.BlockSpec((_TCR, D), lambda i: (i, 0)),
    out_shape=jax.ShapeDtypeStruct((N, D), jnp.float32),
)


_M30 = np.arange(30, dtype=np.float32)
_LGAMMA = np.array(
    [[math.lgamma(m + 1.0) + math.lgamma(m + k + 1.0) for m in range(30)]
     for k in range(K)], dtype=np.float32)


def _coeffs(t):
    lt = jnp.log(t / 2.0)
    cs = []
    for k in range(K):
        ik = jnp.sum(jnp.exp((2.0 * _M30 + k) * lt - _LGAMMA[k]))
        ck = ik if k == 0 else 2.0 * ((-1.0) ** k) * ik
        cs.append(jnp.reshape(ck.astype(jnp.float32), (1, 1)))
    return cs


def _heat_sweep(g0, out_acc, x0, cr, zeros, dinv, cs):
    g = g0
    tm2 = x0
    tm1 = None
    for k in range(1, K):
        p = _seg(g, cr, zeros).reshape(NC, N, D)
        comb = _combine_first if k == 1 else _combine_rest
        tk, g, out_acc = comb(cs[k], p, tm2, dinv, out_acc)
        tm2, tm1 = (x0, tk) if k == 1 else (tm1, tk)
    return out_acc


def kernel(x, edge_index, theta_direct, theta_heat1, theta_hidden,
           theta_heat2, t):
    row = edge_index[0]
    col = edge_index[1]
    rowp = jnp.concatenate(
        [row.reshape(NW, EPW),
         jnp.full((NW, EPAD), N, jnp.int32)], axis=1).reshape(NW, NCHUNK, 1,
                                                              CH)
    colp = jnp.concatenate(
        [col.reshape(NW, EPW),
         jnp.zeros((NW, EPAD), jnp.int32)], axis=1).reshape(NW, NCHUNK, 1, CH)
    cr = jnp.concatenate([colp, rowp], axis=2)
    zeros = jnp.zeros((CH, D), jnp.float32)
    ones = jnp.ones((N, D), jnp.float32)
    cs = _coeffs(t)

    pdeg = _seg(ones, cr, zeros).reshape(NC, N, D)
    dinv, g0, out1 = _prep(cs[0], pdeg, x)
    x_heat = _heat_sweep(g0, out1, x, cr, zeros, dinv, cs)

    hidden, gh0, out2 = _mid(cs[0], x, x_heat, theta_direct, theta_heat1,
                             dinv)
    hidden_heat = _heat_sweep(gh0, out2, hidden, cr, zeros, dinv, cs)

    return _final(hidden, hidden_heat, theta_hidden, theta_heat2)


# speedup vs baseline: 1.9063x; 1.0366x over previous
import functools
import math

import jax
import jax.numpy as jnp
import numpy as np
from jax import lax
from jax.experimental import pallas as pl
from jax.experimental.pallas import tpu as pltpu
from jax.experimental.pallas import tpu_sc as plsc

N = 10000
E = 320000
D = 128
K = 10

NC = 2
NS = 16
NW = NC * NS
EPW = E // NW
CH = 112
NCHUNK = 90
EPAD = NCHUNK * CH - EPW
NACC = 10240
RPS = NACC // NS

HOP = 64
_TCR = 1000
_GRID = N // _TCR


def _seg_body(v_hbm, cr_hbm, zero_hbm, p_hbm, colv0, colv1, colv2,
              rowv0, rowv1, rowv2, gbuf, gbuf1, gbuf2, acc,
              gsem0, gsem1, gsem2, ssem0, ssem1, ssem2):
    c = lax.axis_index("c")
    s = lax.axis_index("s")
    wid = c * NS + s
    colv = (colv0, colv1, colv2)
    rowv = (rowv0, rowv1, rowv2)
    gbufs = (gbuf, gbuf1, gbuf2)
    gsems = (gsem0, gsem1, gsem2)
    ssems = (ssem0, ssem1, ssem2)

    pltpu.sync_copy(zero_hbm, gbuf)
    rbase = pl.multiple_of(s * RPS, 8)
    for h in range(RPS // HOP):
        pltpu.sync_copy(gbuf.at[pl.ds(0, HOP)],
                        acc.at[pl.ds(rbase + h * HOP, HOP)])
    plsc.subcore_barrier()

    pltpu.sync_copy(cr_hbm.at[wid, 0, 0], colv0)
    pltpu.sync_copy(cr_hbm.at[wid, 0, 1], rowv0)
    pltpu.async_copy(v_hbm.at[colv0], gbuf, gsem0)

    def trip(p, carry):
        for i in range(3):
            j = p * 3 + i
            nb = (i + 1) % 3

            @pl.when(j + 1 < NCHUNK)
            def _():
                @pl.when(j >= 2)
                def _():
                    pltpu.make_async_copy(zero_hbm, gbufs[nb],
                                          ssems[nb]).wait()
                pltpu.sync_copy(cr_hbm.at[wid, j + 1, 0], colv[nb])
                pltpu.sync_copy(cr_hbm.at[wid, j + 1, 1], rowv[nb])
                pltpu.async_copy(v_hbm.at[colv[nb]], gbufs[nb], gsems[nb])

            pltpu.make_async_copy(zero_hbm, gbufs[i], gsems[i]).wait()
            pltpu.async_copy(gbufs[i], acc.at[rowv[i]], ssems[i], add=True)
        return carry

    lax.fori_loop(0, NCHUNK // 3, trip, 0)
    for b in range(3):
        pltpu.make_async_copy(zero_hbm, gbufs[b], ssems[b]).wait()
    plsc.subcore_barrier()

    nh = jnp.where(s == NS - 1, (N - (NS - 1) * RPS) // HOP, RPS // HOP)

    def ohop(h, carry):
        rb = pl.multiple_of(rbase + h * HOP, 8)
        pltpu.sync_copy(acc.at[pl.ds(rb, HOP)], gbuf.at[pl.ds(0, HOP)])
        pltpu.sync_copy(gbuf.at[pl.ds(0, HOP)],
                        p_hbm.at[pl.ds(c * N + rb, HOP)])
        return carry

    lax.fori_loop(0, nh, ohop, 0)

    _TB = (NS - 1) * RPS + ((N - (NS - 1) * RPS) // HOP) * HOP
    if N > _TB:
        @pl.when(s == NS - 1)
        def _():
            pltpu.sync_copy(acc.at[pl.ds(_TB, N - _TB)],
                            gbuf.at[pl.ds(0, N - _TB)])
            pltpu.sync_copy(gbuf.at[pl.ds(0, N - _TB)],
                            p_hbm.at[pl.ds(c * N + _TB, N - _TB)])


_seg = pl.kernel(
    _seg_body,
    out_type=jax.ShapeDtypeStruct((NC * N, D), jnp.float32),
    mesh=plsc.VectorSubcoreMesh(core_axis_name="c", subcore_axis_name="s"),
    scratch_types=(
        [pltpu.VMEM((CH,), jnp.int32)] * 3
        + [pltpu.VMEM((CH,), jnp.int32)] * 3
        + [pltpu.VMEM((CH, D), jnp.float32)] * 3
        + [pltpu.VMEM_SHARED((NACC, D), jnp.float32)]
        + [pltpu.SemaphoreType.DMA] * 6
    ),
)


def _prep_body(c0_ref, p_ref, x_ref, dinv_ref, g_ref, out_ref):
    s = p_ref[0] + p_ref[1]
    dinv = jnp.where(s > 0, lax.rsqrt(jnp.maximum(s, 1e-12)), 0.0)
    x = x_ref[...]
    dinv_ref[...] = dinv
    g_ref[...] = dinv * x
    out_ref[...] = c0_ref[0, 0] * x


_prep = pl.pallas_call(
    _prep_body,
    grid=(_GRID,),
    in_specs=[
        pl.BlockSpec(memory_space=pltpu.SMEM),
        pl.BlockSpec((2, _TCR, D), lambda i: (0, i, 0)),
        pl.BlockSpec((_TCR, D), lambda i: (i, 0)),
    ],
    out_specs=[
        pl.BlockSpec((_TCR, D), lambda i: (i, 0)),
        pl.BlockSpec((_TCR, D), lambda i: (i, 0)),
        pl.BlockSpec((_TCR, D), lambda i: (i, 0)),
    ],
    out_shape=[jax.ShapeDtypeStruct((N, D), jnp.float32)] * 3,
)


def _combine_body(ck_ref, p_ref, tm2_ref, dinv_ref, outin_ref,
                  t_ref, g_ref, outnew_ref, *, first):
    s = p_ref[0] + p_ref[1]
    dinv = dinv_ref[...]
    if first:
        t = -dinv * s
    else:
        t = -2.0 * (dinv * s) - tm2_ref[...]
    t_ref[...] = t
    g_ref[...] = dinv * t
    outnew_ref[...] = outin_ref[...] + ck_ref[0, 0] * t


def _make_combine(first):
    return pl.pallas_call(
        functools.partial(_combine_body, first=first),
        grid=(_GRID,),
        in_specs=[
            pl.BlockSpec(memory_space=pltpu.SMEM),
            pl.BlockSpec((2, _TCR, D), lambda i: (0, i, 0)),
            pl.BlockSpec((_TCR, D), lambda i: (i, 0)),
            pl.BlockSpec((_TCR, D), lambda i: (i, 0)),
            pl.BlockSpec((_TCR, D), lambda i: (i, 0)),
        ],
        out_specs=[
            pl.BlockSpec((_TCR, D), lambda i: (i, 0)),
            pl.BlockSpec((_TCR, D), lambda i: (i, 0)),
            pl.BlockSpec((_TCR, D), lambda i: (i, 0)),
        ],
        out_shape=[jax.ShapeDtypeStruct((N, D), jnp.float32)] * 3,
    )


_combine_first = _make_combine(True)
_combine_rest = _make_combine(False)


def _mid_body(c0_ref, x_ref, xh_ref, td_ref, th1_ref, dinv_ref,
              hid_ref, g_ref, out_ref):
    h = jnp.dot(x_ref[...], td_ref[...], preferred_element_type=jnp.float32)
    h += jnp.dot(xh_ref[...], th1_ref[...], preferred_element_type=jnp.float32)
    h = jnp.maximum(h, 0.0)
    hid_ref[...] = h
    g_ref[...] = dinv_ref[...] * h
    out_ref[...] = c0_ref[0, 0] * h


_mid = pl.pallas_call(
    _mid_body,
    grid=(_GRID,),
    in_specs=[
        pl.BlockSpec(memory_space=pltpu.SMEM),
        pl.BlockSpec((_TCR, D), lambda i: (i, 0)),
        pl.BlockSpec((_TCR, D), lambda i: (i, 0)),
        pl.BlockSpec((D, D), lambda i: (0, 0)),
        pl.BlockSpec((D, D), lambda i: (0, 0)),
        pl.BlockSpec((_TCR, D), lambda i: (i, 0)),
    ],
    out_specs=[
        pl.BlockSpec((_TCR, D), lambda i: (i, 0)),
        pl.BlockSpec((_TCR, D), lambda i: (i, 0)),
        pl.BlockSpec((_TCR, D), lambda i: (i, 0)),
    ],
    out_shape=[jax.ShapeDtypeStruct((N, D), jnp.float32)] * 3,
)


def _final_body(h_ref, hh_ref, th_ref, th2_ref, o_ref):
    z = jnp.dot(h_ref[...], th_ref[...], preferred_element_type=jnp.float32)
    z += jnp.dot(hh_ref[...], th2_ref[...], preferred_element_type=jnp.float32)
    m = jnp.max(z, axis=1, keepdims=True)
    lse = m + jnp.log(jnp.sum(jnp.exp(z - m), axis=1, keepdims=True))
    o_ref[...] = z - lse


_final = pl.pallas_call(
    _final_body,
    grid=(_GRID,),
    in_specs=[
        pl.BlockSpec((_TCR, D), lambda i: (i, 0)),
        pl.BlockSpec((_TCR, D), lambda i: (i, 0)),
        pl.BlockSpec((D, D), lambda i: (0, 0)),
        pl.BlockSpec((D, D), lambda i: (0, 0)),
    ],
    out_specs=pl.BlockSpec((_TCR, D), lambda i: (i, 0)),
    out_shape=jax.ShapeDtypeStruct((N, D), jnp.float32),
)


_M30 = np.arange(30, dtype=np.float32)
_LGAMMA = np.array(
    [[math.lgamma(m + 1.0) + math.lgamma(m + k + 1.0) for m in range(30)]
     for k in range(K)], dtype=np.float32)


def _coeffs(t):
    lt = jnp.log(t / 2.0)
    cs = []
    for k in range(K):
        ik = jnp.sum(jnp.exp((2.0 * _M30 + k) * lt - _LGAMMA[k]))
        ck = ik if k == 0 else 2.0 * ((-1.0) ** k) * ik
        cs.append(jnp.reshape(ck.astype(jnp.float32), (1, 1)))
    return cs


def _heat_sweep(g0, out_acc, x0, cr, zeros, dinv, cs):
    g = g0
    tm2 = x0
    tm1 = None
    for k in range(1, K):
        p = _seg(g, cr, zeros).reshape(NC, N, D)
        comb = _combine_first if k == 1 else _combine_rest
        tk, g, out_acc = comb(cs[k], p, tm2, dinv, out_acc)
        tm2, tm1 = (x0, tk) if k == 1 else (tm1, tk)
    return out_acc


def kernel(x, edge_index, theta_direct, theta_heat1, theta_hidden,
           theta_heat2, t):
    row = edge_index[0]
    col = edge_index[1]
    rowp = jnp.concatenate(
        [row.reshape(NW, EPW),
         jnp.full((NW, EPAD), N, jnp.int32)], axis=1).reshape(NW, NCHUNK, 1,
                                                              CH)
    colp = jnp.concatenate(
        [col.reshape(NW, EPW),
         jnp.zeros((NW, EPAD), jnp.int32)], axis=1).reshape(NW, NCHUNK, 1, CH)
    cr = jnp.concatenate([colp, rowp], axis=2)
    zeros = jnp.zeros((CH, D), jnp.float32)
    ones = jnp.ones((N, D), jnp.float32)
    cs = _coeffs(t)

    pdeg = _seg(ones, cr, zeros).reshape(NC, N, D)
    dinv, g0, out1 = _prep(cs[0], pdeg, x)
    x_heat = _heat_sweep(g0, out1, x, cr, zeros, dinv, cs)

    hidden, gh0, out2 = _mid(cs[0], x, x_heat, theta_direct, theta_heat1,
                             dinv)
    hidden_heat = _heat_sweep(gh0, out2, hidden, cr, zeros, dinv, cs)

    return _final(hidden, hidden_heat, theta_hidden, theta_heat2)


# speedup vs baseline: 1.9172x; 1.0057x over previous
import functools
import math

import jax
import jax.numpy as jnp
import numpy as np
from jax import lax
from jax.experimental import pallas as pl
from jax.experimental.pallas import tpu as pltpu
from jax.experimental.pallas import tpu_sc as plsc

N = 10000
E = 320000
D = 128
K = 10

NC = 2
NS = 16
NW = NC * NS
EPW = E // NW
CH = 120
NCHUNK = 84
EPAD = NCHUNK * CH - EPW
NACC = 10240
RPS = NACC // NS

HOP = 64
_TCR = 1000
_GRID = N // _TCR


def _seg_body(v_hbm, cr_hbm, zero_hbm, p_hbm, colv0, colv1, colv2,
              rowv0, rowv1, rowv2, gbuf, gbuf1, gbuf2, acc,
              gsem0, gsem1, gsem2, ssem0, ssem1, ssem2):
    c = lax.axis_index("c")
    s = lax.axis_index("s")
    wid = c * NS + s
    colv = (colv0, colv1, colv2)
    rowv = (rowv0, rowv1, rowv2)
    gbufs = (gbuf, gbuf1, gbuf2)
    gsems = (gsem0, gsem1, gsem2)
    ssems = (ssem0, ssem1, ssem2)

    pltpu.sync_copy(zero_hbm, gbuf)
    rbase = pl.multiple_of(s * RPS, 8)
    for h in range(RPS // HOP):
        pltpu.sync_copy(gbuf.at[pl.ds(0, HOP)],
                        acc.at[pl.ds(rbase + h * HOP, HOP)])
    plsc.subcore_barrier()

    pltpu.sync_copy(cr_hbm.at[wid, 0, 0], colv0)
    pltpu.sync_copy(cr_hbm.at[wid, 0, 1], rowv0)
    pltpu.async_copy(v_hbm.at[colv0], gbuf, gsem0)

    def trip(p, carry):
        for i in range(3):
            j = p * 3 + i
            nb = (i + 1) % 3

            @pl.when(j + 1 < NCHUNK)
            def _():
                @pl.when(j >= 2)
                def _():
                    pltpu.make_async_copy(zero_hbm, gbufs[nb],
                                          ssems[nb]).wait()
                pltpu.sync_copy(cr_hbm.at[wid, j + 1, 0], colv[nb])
                pltpu.sync_copy(cr_hbm.at[wid, j + 1, 1], rowv[nb])
                pltpu.async_copy(v_hbm.at[colv[nb]], gbufs[nb], gsems[nb])

            pltpu.make_async_copy(zero_hbm, gbufs[i], gsems[i]).wait()
            pltpu.async_copy(gbufs[i], acc.at[rowv[i]], ssems[i], add=True)
        return carry

    lax.fori_loop(0, NCHUNK // 3, trip, 0)
    for b in range(3):
        pltpu.make_async_copy(zero_hbm, gbufs[b], ssems[b]).wait()
    plsc.subcore_barrier()

    nh = jnp.where(s == NS - 1, (N - (NS - 1) * RPS) // HOP, RPS // HOP)

    def ohop(h, carry):
        rb = pl.multiple_of(rbase + h * HOP, 8)
        pltpu.sync_copy(acc.at[pl.ds(rb, HOP)], gbuf.at[pl.ds(0, HOP)])
        pltpu.sync_copy(gbuf.at[pl.ds(0, HOP)],
                        p_hbm.at[pl.ds(c * N + rb, HOP)])
        return carry

    lax.fori_loop(0, nh, ohop, 0)

    _TB = (NS - 1) * RPS + ((N - (NS - 1) * RPS) // HOP) * HOP
    if N > _TB:
        @pl.when(s == NS - 1)
        def _():
            pltpu.sync_copy(acc.at[pl.ds(_TB, N - _TB)],
                            gbuf.at[pl.ds(0, N - _TB)])
            pltpu.sync_copy(gbuf.at[pl.ds(0, N - _TB)],
                            p_hbm.at[pl.ds(c * N + _TB, N - _TB)])


_seg = pl.kernel(
    _seg_body,
    out_type=jax.ShapeDtypeStruct((NC * N, D), jnp.float32),
    mesh=plsc.VectorSubcoreMesh(core_axis_name="c", subcore_axis_name="s"),
    scratch_types=(
        [pltpu.VMEM((CH,), jnp.int32)] * 3
        + [pltpu.VMEM((CH,), jnp.int32)] * 3
        + [pltpu.VMEM((CH, D), jnp.float32)] * 3
        + [pltpu.VMEM_SHARED((NACC, D), jnp.float32)]
        + [pltpu.SemaphoreType.DMA] * 6
    ),
)


def _prep_body(c0_ref, p_ref, x_ref, dinv_ref, g_ref, out_ref):
    s = p_ref[0] + p_ref[1]
    dinv = jnp.where(s > 0, lax.rsqrt(jnp.maximum(s, 1e-12)), 0.0)
    x = x_ref[...]
    dinv_ref[...] = dinv
    g_ref[...] = dinv * x
    out_ref[...] = c0_ref[0, 0] * x


_prep = pl.pallas_call(
    _prep_body,
    grid=(_GRID,),
    in_specs=[
        pl.BlockSpec(memory_space=pltpu.SMEM),
        pl.BlockSpec((2, _TCR, D), lambda i: (0, i, 0)),
        pl.BlockSpec((_TCR, D), lambda i: (i, 0)),
    ],
    out_specs=[
        pl.BlockSpec((_TCR, D), lambda i: (i, 0)),
        pl.BlockSpec((_TCR, D), lambda i: (i, 0)),
        pl.BlockSpec((_TCR, D), lambda i: (i, 0)),
    ],
    out_shape=[jax.ShapeDtypeStruct((N, D), jnp.float32)] * 3,
)


def _combine_body(ck_ref, p_ref, tm2_ref, dinv_ref, outin_ref,
                  t_ref, g_ref, outnew_ref, *, first):
    s = p_ref[0] + p_ref[1]
    dinv = dinv_ref[...]
    if first:
        t = -dinv * s
    else:
        t = -2.0 * (dinv * s) - tm2_ref[...]
    t_ref[...] = t
    g_ref[...] = dinv * t
    outnew_ref[...] = outin_ref[...] + ck_ref[0, 0] * t


def _make_combine(first):
    return pl.pallas_call(
        functools.partial(_combine_body, first=first),
        grid=(_GRID,),
        in_specs=[
            pl.BlockSpec(memory_space=pltpu.SMEM),
            pl.BlockSpec((2, _TCR, D), lambda i: (0, i, 0)),
            pl.BlockSpec((_TCR, D), lambda i: (i, 0)),
            pl.BlockSpec((_TCR, D), lambda i: (i, 0)),
            pl.BlockSpec((_TCR, D), lambda i: (i, 0)),
        ],
        out_specs=[
            pl.BlockSpec((_TCR, D), lambda i: (i, 0)),
            pl.BlockSpec((_TCR, D), lambda i: (i, 0)),
            pl.BlockSpec((_TCR, D), lambda i: (i, 0)),
        ],
        out_shape=[jax.ShapeDtypeStruct((N, D), jnp.float32)] * 3,
    )


_combine_first = _make_combine(True)
_combine_rest = _make_combine(False)


def _mid_body(c0_ref, x_ref, xh_ref, td_ref, th1_ref, dinv_ref,
              hid_ref, g_ref, out_ref):
    h = jnp.dot(x_ref[...], td_ref[...], preferred_element_type=jnp.float32)
    h += jnp.dot(xh_ref[...], th1_ref[...], preferred_element_type=jnp.float32)
    h = jnp.maximum(h, 0.0)
    hid_ref[...] = h
    g_ref[...] = dinv_ref[...] * h
    out_ref[...] = c0_ref[0, 0] * h


_mid = pl.pallas_call(
    _mid_body,
    grid=(_GRID,),
    in_specs=[
        pl.BlockSpec(memory_space=pltpu.SMEM),
        pl.BlockSpec((_TCR, D), lambda i: (i, 0)),
        pl.BlockSpec((_TCR, D), lambda i: (i, 0)),
        pl.BlockSpec((D, D), lambda i: (0, 0)),
        pl.BlockSpec((D, D), lambda i: (0, 0)),
        pl.BlockSpec((_TCR, D), lambda i: (i, 0)),
    ],
    out_specs=[
        pl.BlockSpec((_TCR, D), lambda i: (i, 0)),
        pl.BlockSpec((_TCR, D), lambda i: (i, 0)),
        pl.BlockSpec((_TCR, D), lambda i: (i, 0)),
    ],
    out_shape=[jax.ShapeDtypeStruct((N, D), jnp.float32)] * 3,
)


def _final_body(h_ref, hh_ref, th_ref, th2_ref, o_ref):
    z = jnp.dot(h_ref[...], th_ref[...], preferred_element_type=jnp.float32)
    z += jnp.dot(hh_ref[...], th2_ref[...], preferred_element_type=jnp.float32)
    m = jnp.max(z, axis=1, keepdims=True)
    lse = m + jnp.log(jnp.sum(jnp.exp(z - m), axis=1, keepdims=True))
    o_ref[...] = z - lse


_final = pl.pallas_call(
    _final_body,
    grid=(_GRID,),
    in_specs=[
        pl.BlockSpec((_TCR, D), lambda i: (i, 0)),
        pl.BlockSpec((_TCR, D), lambda i: (i, 0)),
        pl.BlockSpec((D, D), lambda i: (0, 0)),
        pl.BlockSpec((D, D), lambda i: (0, 0)),
    ],
    out_specs=pl.BlockSpec((_TCR, D), lambda i: (i, 0)),
    out_shape=jax.ShapeDtypeStruct((N, D), jnp.float32),
)


_M30 = np.arange(30, dtype=np.float32)
_LGAMMA = np.array(
    [[math.lgamma(m + 1.0) + math.lgamma(m + k + 1.0) for m in range(30)]
     for k in range(K)], dtype=np.float32)


def _coeffs(t):
    lt = jnp.log(t / 2.0)
    cs = []
    for k in range(K):
        ik = jnp.sum(jnp.exp((2.0 * _M30 + k) * lt - _LGAMMA[k]))
        ck = ik if k == 0 else 2.0 * ((-1.0) ** k) * ik
        cs.append(jnp.reshape(ck.astype(jnp.float32), (1, 1)))
    return cs


def _heat_sweep(g0, out_acc, x0, cr, zeros, dinv, cs):
    g = g0
    tm2 = x0
    tm1 = None
    for k in range(1, K):
        p = _seg(g, cr, zeros).reshape(NC, N, D)
        comb = _combine_first if k == 1 else _combine_rest
        tk, g, out_acc = comb(cs[k], p, tm2, dinv, out_acc)
        tm2, tm1 = (x0, tk) if k == 1 else (tm1, tk)
    return out_acc


def kernel(x, edge_index, theta_direct, theta_heat1, theta_hidden,
           theta_heat2, t):
    row = edge_index[0]
    col = edge_index[1]
    rowp = jnp.concatenate(
        [row.reshape(NW, EPW),
         jnp.full((NW, EPAD), N, jnp.int32)], axis=1).reshape(NW, NCHUNK, 1,
                                                              CH)
    colp = jnp.concatenate(
        [col.reshape(NW, EPW),
         jnp.zeros((NW, EPAD), jnp.int32)], axis=1).reshape(NW, NCHUNK, 1, CH)
    cr = jnp.concatenate([colp, rowp], axis=2)
    zeros = jnp.zeros((CH, D), jnp.float32)
    ones = jnp.ones((N, D), jnp.float32)
    cs = _coeffs(t)

    pdeg = _seg(ones, cr, zeros).reshape(NC, N, D)
    dinv, g0, out1 = _prep(cs[0], pdeg, x)
    x_heat = _heat_sweep(g0, out1, x, cr, zeros, dinv, cs)

    hidden, gh0, out2 = _mid(cs[0], x, x_heat, theta_direct, theta_heat1,
                             dinv)
    hidden_heat = _heat_sweep(gh0, out2, hidden, cr, zeros, dinv, cs)

    return _final(hidden, hidden_heat, theta_hidden, theta_heat2)


# speedup vs baseline: 2.1323x; 1.1122x over previous
import functools
import math

import jax
import jax.numpy as jnp
import numpy as np
from jax import lax
from jax.experimental import pallas as pl
from jax.experimental.pallas import tpu as pltpu
from jax.experimental.pallas import tpu_sc as plsc

N = 10000
E = 320000
D = 128
K = 10

NC = 2
NS = 16
NW = NC * NS
EPW = E // NW
CH = 120
NCHUNK = 84
EPAD = NCHUNK * CH - EPW
NACC = 10240
RPS = NACC // NS

HOP = 64
_TCR = 1000
_GRID = N // _TCR


def _seg_body(v_hbm, cr_hbm, zero_hbm, p_hbm,
              colv0, colv1, colv2, colv3, rowv0, rowv1, rowv2, rowv3,
              gbuf, gbuf1, gbuf2, acc,
              isem0, isem1, isem2, isem3,
              gsem0, gsem1, gsem2, ssem0, ssem1, ssem2):
    c = lax.axis_index("c")
    s = lax.axis_index("s")
    wid = c * NS + s
    colv = (colv0, colv1, colv2, colv3)
    rowv = (rowv0, rowv1, rowv2, rowv3)
    gbufs = (gbuf, gbuf1, gbuf2)
    isems = (isem0, isem1, isem2, isem3)
    gsems = (gsem0, gsem1, gsem2)
    ssems = (ssem0, ssem1, ssem2)

    pltpu.sync_copy(zero_hbm, gbuf)
    rbase = pl.multiple_of(s * RPS, 8)
    for h in range(RPS // HOP):
        pltpu.sync_copy(gbuf.at[pl.ds(0, HOP)],
                        acc.at[pl.ds(rbase + h * HOP, HOP)])
    plsc.subcore_barrier()

    def _ifire(j, b):
        pltpu.async_copy(cr_hbm.at[wid, j, 0], colv[b], isems[b])
        pltpu.async_copy(cr_hbm.at[wid, j, 1], rowv[b], isems[b])

    def _iwait(b):
        pltpu.make_async_copy(cr_hbm.at[0, 0, 0], colv[b], isems[b]).wait()
        pltpu.make_async_copy(cr_hbm.at[0, 0, 1], rowv[b], isems[b]).wait()

    def _swait(b):
        pltpu.make_async_copy(zero_hbm, gbufs[b], ssems[b]).wait()

    def _gwait(b):
        pltpu.make_async_copy(zero_hbm, gbufs[b], gsems[b]).wait()

    _ifire(0, 0)
    _ifire(1, 1)
    _iwait(0)
    pltpu.async_copy(v_hbm.at[colv[0]], gbufs[0], gsems[0])

    def block(p, carry):
        for i in range(12):
            j = p * 12 + i

            @pl.when(j + 1 < NCHUNK)
            def _():
                nb = (i + 1) % 3

                @pl.when(j >= 2)
                def _():
                    _swait(nb)
                _iwait((i + 1) % 4)
                pltpu.async_copy(v_hbm.at[colv[(i + 1) % 4]], gbufs[nb],
                                 gsems[nb])

            @pl.when(j + 2 < NCHUNK)
            def _():
                _ifire(j + 2, (i + 2) % 4)

            _gwait(i % 3)
            pltpu.async_copy(gbufs[i % 3], acc.at[rowv[i % 4]], ssems[i % 3],
                             add=True)
        return carry

    lax.fori_loop(0, NCHUNK // 12, block, 0)
    for b in range(3):
        _swait(b)
    plsc.subcore_barrier()

    nh = jnp.where(s == NS - 1, (N - (NS - 1) * RPS) // HOP, RPS // HOP)

    def ohop(h, carry):
        rb = pl.multiple_of(rbase + h * HOP, 8)
        pltpu.sync_copy(acc.at[pl.ds(rb, HOP)], gbuf.at[pl.ds(0, HOP)])
        pltpu.sync_copy(gbuf.at[pl.ds(0, HOP)],
                        p_hbm.at[pl.ds(c * N + rb, HOP)])
        return carry

    lax.fori_loop(0, nh, ohop, 0)

    _TB = (NS - 1) * RPS + ((N - (NS - 1) * RPS) // HOP) * HOP
    if N > _TB:
        @pl.when(s == NS - 1)
        def _():
            pltpu.sync_copy(acc.at[pl.ds(_TB, N - _TB)],
                            gbuf.at[pl.ds(0, N - _TB)])
            pltpu.sync_copy(gbuf.at[pl.ds(0, N - _TB)],
                            p_hbm.at[pl.ds(c * N + _TB, N - _TB)])


_seg = pl.kernel(
    _seg_body,
    out_type=jax.ShapeDtypeStruct((NC * N, D), jnp.float32),
    mesh=plsc.VectorSubcoreMesh(core_axis_name="c", subcore_axis_name="s"),
    scratch_types=(
        [pltpu.VMEM((CH,), jnp.int32)] * 4
        + [pltpu.VMEM((CH,), jnp.int32)] * 4
        + [pltpu.VMEM((CH, D), jnp.float32)] * 3
        + [pltpu.VMEM_SHARED((NACC, D), jnp.float32)]
        + [pltpu.SemaphoreType.DMA] * 10
    ),
)


def _prep_body(c0_ref, p_ref, x_ref, dinv_ref, g_ref, out_ref):
    s = p_ref[0] + p_ref[1]
    dinv = jnp.where(s > 0, lax.rsqrt(jnp.maximum(s, 1e-12)), 0.0)
    x = x_ref[...]
    dinv_ref[...] = dinv
    g_ref[...] = dinv * x
    out_ref[...] = c0_ref[0, 0] * x


_prep = pl.pallas_call(
    _prep_body,
    grid=(_GRID,),
    in_specs=[
        pl.BlockSpec(memory_space=pltpu.SMEM),
        pl.BlockSpec((2, _TCR, D), lambda i: (0, i, 0)),
        pl.BlockSpec((_TCR, D), lambda i: (i, 0)),
    ],
    out_specs=[
        pl.BlockSpec((_TCR, D), lambda i: (i, 0)),
        pl.BlockSpec((_TCR, D), lambda i: (i, 0)),
        pl.BlockSpec((_TCR, D), lambda i: (i, 0)),
    ],
    out_shape=[jax.ShapeDtypeStruct((N, D), jnp.float32)] * 3,
)


def _combine_body(ck_ref, p_ref, tm2_ref, dinv_ref, outin_ref,
                  t_ref, g_ref, outnew_ref, *, first):
    s = p_ref[0] + p_ref[1]
    dinv = dinv_ref[...]
    if first:
        t = -dinv * s
    else:
        t = -2.0 * (dinv * s) - tm2_ref[...]
    t_ref[...] = t
    g_ref[...] = dinv * t
    outnew_ref[...] = outin_ref[...] + ck_ref[0, 0] * t


def _make_combine(first):
    return pl.pallas_call(
        functools.partial(_combine_body, first=first),
        grid=(_GRID,),
        in_specs=[
            pl.BlockSpec(memory_space=pltpu.SMEM),
            pl.BlockSpec((2, _TCR, D), lambda i: (0, i, 0)),
            pl.BlockSpec((_TCR, D), lambda i: (i, 0)),
            pl.BlockSpec((_TCR, D), lambda i: (i, 0)),
            pl.BlockSpec((_TCR, D), lambda i: (i, 0)),
        ],
        out_specs=[
            pl.BlockSpec((_TCR, D), lambda i: (i, 0)),
            pl.BlockSpec((_TCR, D), lambda i: (i, 0)),
            pl.BlockSpec((_TCR, D), lambda i: (i, 0)),
        ],
        out_shape=[jax.ShapeDtypeStruct((N, D), jnp.float32)] * 3,
    )


_combine_first = _make_combine(True)
_combine_rest = _make_combine(False)


def _mid_body(c0_ref, x_ref, xh_ref, td_ref, th1_ref, dinv_ref,
              hid_ref, g_ref, out_ref):
    h = jnp.dot(x_ref[...], td_ref[...], preferred_element_type=jnp.float32)
    h += jnp.dot(xh_ref[...], th1_ref[...], preferred_element_type=jnp.float32)
    h = jnp.maximum(h, 0.0)
    hid_ref[...] = h
    g_ref[...] = dinv_ref[...] * h
    out_ref[...] = c0_ref[0, 0] * h


_mid = pl.pallas_call(
    _mid_body,
    grid=(_GRID,),
    in_specs=[
        pl.BlockSpec(memory_space=pltpu.SMEM),
        pl.BlockSpec((_TCR, D), lambda i: (i, 0)),
        pl.BlockSpec((_TCR, D), lambda i: (i, 0)),
        pl.BlockSpec((D, D), lambda i: (0, 0)),
        pl.BlockSpec((D, D), lambda i: (0, 0)),
        pl.BlockSpec((_TCR, D), lambda i: (i, 0)),
    ],
    out_specs=[
        pl.BlockSpec((_TCR, D), lambda i: (i, 0)),
        pl.BlockSpec((_TCR, D), lambda i: (i, 0)),
        pl.BlockSpec((_TCR, D), lambda i: (i, 0)),
    ],
    out_shape=[jax.ShapeDtypeStruct((N, D), jnp.float32)] * 3,
)


def _final_body(h_ref, hh_ref, th_ref, th2_ref, o_ref):
    z = jnp.dot(h_ref[...], th_ref[...], preferred_element_type=jnp.float32)
    z += jnp.dot(hh_ref[...], th2_ref[...], preferred_element_type=jnp.float32)
    m = jnp.max(z, axis=1, keepdims=True)
    lse = m + jnp.log(jnp.sum(jnp.exp(z - m), axis=1, keepdims=True))
    o_ref[...] = z - lse


_final = pl.pallas_call(
    _final_body,
    grid=(_GRID,),
    in_specs=[
        pl.BlockSpec((_TCR, D), lambda i: (i, 0)),
        pl.BlockSpec((_TCR, D), lambda i: (i, 0)),
        pl.BlockSpec((D, D), lambda i: (0, 0)),
        pl.BlockSpec((D, D), lambda i: (0, 0)),
    ],
    out_specs=pl.BlockSpec((_TCR, D), lambda i: (i, 0)),
    out_shape=jax.ShapeDtypeStruct((N, D), jnp.float32),
)


_M30 = np.arange(30, dtype=np.float32)
_LGAMMA = np.array(
    [[math.lgamma(m + 1.0) + math.lgamma(m + k + 1.0) for m in range(30)]
     for k in range(K)], dtype=np.float32)


def _coeffs(t):
    lt = jnp.log(t / 2.0)
    cs = []
    for k in range(K):
        ik = jnp.sum(jnp.exp((2.0 * _M30 + k) * lt - _LGAMMA[k]))
        ck = ik if k == 0 else 2.0 * ((-1.0) ** k) * ik
        cs.append(jnp.reshape(ck.astype(jnp.float32), (1, 1)))
    return cs


def _heat_sweep(g0, out_acc, x0, cr, zeros, dinv, cs):
    g = g0
    tm2 = x0
    tm1 = None
    for k in range(1, K):
        p = _seg(g, cr, zeros).reshape(NC, N, D)
        comb = _combine_first if k == 1 else _combine_rest
        tk, g, out_acc = comb(cs[k], p, tm2, dinv, out_acc)
        tm2, tm1 = (x0, tk) if k == 1 else (tm1, tk)
    return out_acc


def kernel(x, edge_index, theta_direct, theta_heat1, theta_hidden,
           theta_heat2, t):
    row = edge_index[0]
    col = edge_index[1]
    rowp = jnp.concatenate(
        [row.reshape(NW, EPW),
         jnp.full((NW, EPAD), N, jnp.int32)], axis=1).reshape(NW, NCHUNK, 1,
                                                              CH)
    colp = jnp.concatenate(
        [col.reshape(NW, EPW),
         jnp.zeros((NW, EPAD), jnp.int32)], axis=1).reshape(NW, NCHUNK, 1, CH)
    cr = jnp.concatenate([colp, rowp], axis=2)
    zeros = jnp.zeros((CH, D), jnp.float32)
    ones = jnp.ones((N, D), jnp.float32)
    cs = _coeffs(t)

    pdeg = _seg(ones, cr, zeros).reshape(NC, N, D)
    dinv, g0, out1 = _prep(cs[0], pdeg, x)
    x_heat = _heat_sweep(g0, out1, x, cr, zeros, dinv, cs)

    hidden, gh0, out2 = _mid(cs[0], x, x_heat, theta_direct, theta_heat1,
                             dinv)
    hidden_heat = _heat_sweep(gh0, out2, hidden, cr, zeros, dinv, cs)

    return _final(hidden, hidden_heat, theta_hidden, theta_heat2)


# speedup vs baseline: 2.1633x; 1.0146x over previous
import functools
import math

import jax
import jax.numpy as jnp
import numpy as np
from jax import lax
from jax.experimental import pallas as pl
from jax.experimental.pallas import tpu as pltpu
from jax.experimental.pallas import tpu_sc as plsc

N = 10000
E = 320000
D = 128
K = 10

NC = 2
NS = 16
NW = NC * NS
EPW = E // NW
CH = 120
NCHUNK = 84
EPAD = NCHUNK * CH - EPW
NACC = 10240
RPS = NACC // NS

HOP = 64
_TCR = 1000
_GRID = N // _TCR


def _seg_body(v_hbm, cr_hbm, zero_hbm, p_hbm,
              colv0, colv1, colv2, colv3, rowv0, rowv1, rowv2, rowv3,
              gbuf, gbuf1, gbuf2, acc,
              isem0, isem1, isem2, isem3,
              gsem0, gsem1, gsem2, ssem0, ssem1, ssem2):
    c = lax.axis_index("c")
    s = lax.axis_index("s")
    wid = c * NS + s
    colv = (colv0, colv1, colv2, colv3)
    rowv = (rowv0, rowv1, rowv2, rowv3)
    gbufs = (gbuf, gbuf1, gbuf2)
    isems = (isem0, isem1, isem2, isem3)
    gsems = (gsem0, gsem1, gsem2)
    ssems = (ssem0, ssem1, ssem2)

    pltpu.sync_copy(zero_hbm, gbuf)
    rbase = pl.multiple_of(s * RPS, 8)
    for h in range(RPS // HOP):
        pltpu.async_copy(gbuf.at[pl.ds(0, HOP)],
                         acc.at[pl.ds(rbase + h * HOP, HOP)],
                         ssems[h % 3])
    for h in range(RPS // HOP):
        pltpu.make_async_copy(zero_hbm.at[pl.ds(0, HOP)],
                              gbuf.at[pl.ds(0, HOP)], ssems[h % 3]).wait()
    plsc.subcore_barrier()

    def _ifire(j, b):
        pltpu.async_copy(cr_hbm.at[wid, j, 0], colv[b], isems[b])
        pltpu.async_copy(cr_hbm.at[wid, j, 1], rowv[b], isems[b])

    def _iwait(b):
        pltpu.make_async_copy(cr_hbm.at[0, 0, 0], colv[b], isems[b]).wait()
        pltpu.make_async_copy(cr_hbm.at[0, 0, 1], rowv[b], isems[b]).wait()

    def _swait(b):
        pltpu.make_async_copy(zero_hbm, gbufs[b], ssems[b]).wait()

    def _gwait(b):
        pltpu.make_async_copy(zero_hbm, gbufs[b], gsems[b]).wait()

    _ifire(0, 0)
    _ifire(1, 1)
    _iwait(0)
    pltpu.async_copy(v_hbm.at[colv[0]], gbufs[0], gsems[0])

    def block(p, carry):
        for i in range(12):
            j = p * 12 + i

            @pl.when(j + 1 < NCHUNK)
            def _():
                nb = (i + 1) % 3

                @pl.when(j >= 2)
                def _():
                    _swait(nb)
                _iwait((i + 1) % 4)
                pltpu.async_copy(v_hbm.at[colv[(i + 1) % 4]], gbufs[nb],
                                 gsems[nb])

            @pl.when(j + 2 < NCHUNK)
            def _():
                _ifire(j + 2, (i + 2) % 4)

            _gwait(i % 3)
            pltpu.async_copy(gbufs[i % 3], acc.at[rowv[i % 4]], ssems[i % 3],
                             add=True)
        return carry

    lax.fori_loop(0, NCHUNK // 12, block, 0)
    for b in range(3):
        _swait(b)
    plsc.subcore_barrier()

    def _co(h):
        b = h % 3
        if h >= 3:
            pltpu.make_async_copy(zero_hbm.at[pl.ds(0, HOP)],
                                  gbufs[b].at[pl.ds(0, HOP)],
                                  gsems[b]).wait()
        rb = pl.multiple_of(rbase + h * HOP, 8)
        pltpu.sync_copy(acc.at[pl.ds(rb, HOP)], gbufs[b].at[pl.ds(0, HOP)])
        pltpu.async_copy(gbufs[b].at[pl.ds(0, HOP)],
                         p_hbm.at[pl.ds(c * N + rb, HOP)], gsems[b])

    _NHL = (N - (NS - 1) * RPS) // HOP
    for h in range(_NHL):
        _co(h)

    @pl.when(s < NS - 1)
    def _():
        for h in range(_NHL, RPS // HOP):
            _co(h)

    for b in range(3):
        pltpu.make_async_copy(zero_hbm.at[pl.ds(0, HOP)],
                              gbufs[b].at[pl.ds(0, HOP)], gsems[b]).wait()

    _TB = (NS - 1) * RPS + _NHL * HOP
    if N > _TB:
        @pl.when(s == NS - 1)
        def _():
            pltpu.sync_copy(acc.at[pl.ds(_TB, N - _TB)],
                            gbuf.at[pl.ds(0, N - _TB)])
            pltpu.sync_copy(gbuf.at[pl.ds(0, N - _TB)],
                            p_hbm.at[pl.ds(c * N + _TB, N - _TB)])


_seg = pl.kernel(
    _seg_body,
    out_type=jax.ShapeDtypeStruct((NC * N, D), jnp.float32),
    mesh=plsc.VectorSubcoreMesh(core_axis_name="c", subcore_axis_name="s"),
    scratch_types=(
        [pltpu.VMEM((CH,), jnp.int32)] * 4
        + [pltpu.VMEM((CH,), jnp.int32)] * 4
        + [pltpu.VMEM((CH, D), jnp.float32)] * 3
        + [pltpu.VMEM_SHARED((NACC, D), jnp.float32)]
        + [pltpu.SemaphoreType.DMA] * 10
    ),
)


def _prep_body(c0_ref, p_ref, x_ref, dinv_ref, g_ref, out_ref):
    s = p_ref[0] + p_ref[1]
    dinv = jnp.where(s > 0, lax.rsqrt(jnp.maximum(s, 1e-12)), 0.0)
    x = x_ref[...]
    dinv_ref[...] = dinv
    g_ref[...] = dinv * x
    out_ref[...] = c0_ref[0, 0] * x


_prep = pl.pallas_call(
    _prep_body,
    grid=(_GRID,),
    in_specs=[
        pl.BlockSpec(memory_space=pltpu.SMEM),
        pl.BlockSpec((2, _TCR, D), lambda i: (0, i, 0)),
        pl.BlockSpec((_TCR, D), lambda i: (i, 0)),
    ],
    out_specs=[
        pl.BlockSpec((_TCR, D), lambda i: (i, 0)),
        pl.BlockSpec((_TCR, D), lambda i: (i, 0)),
        pl.BlockSpec((_TCR, D), lambda i: (i, 0)),
    ],
    out_shape=[jax.ShapeDtypeStruct((N, D), jnp.float32)] * 3,
)


def _combine_body(ck_ref, p_ref, tm2_ref, dinv_ref, outin_ref,
                  t_ref, g_ref, outnew_ref, *, first):
    s = p_ref[0] + p_ref[1]
    dinv = dinv_ref[...]
    if first:
        t = -dinv * s
    else:
        t = -2.0 * (dinv * s) - tm2_ref[...]
    t_ref[...] = t
    g_ref[...] = dinv * t
    outnew_ref[...] = outin_ref[...] + ck_ref[0, 0] * t


def _make_combine(first):
    return pl.pallas_call(
        functools.partial(_combine_body, first=first),
        grid=(_GRID,),
        in_specs=[
            pl.BlockSpec(memory_space=pltpu.SMEM),
            pl.BlockSpec((2, _TCR, D), lambda i: (0, i, 0)),
            pl.BlockSpec((_TCR, D), lambda i: (i, 0)),
            pl.BlockSpec((_TCR, D), lambda i: (i, 0)),
            pl.BlockSpec((_TCR, D), lambda i: (i, 0)),
        ],
        out_specs=[
            pl.BlockSpec((_TCR, D), lambda i: (i, 0)),
            pl.BlockSpec((_TCR, D), lambda i: (i, 0)),
            pl.BlockSpec((_TCR, D), lambda i: (i, 0)),
        ],
        out_shape=[jax.ShapeDtypeStruct((N, D), jnp.float32)] * 3,
    )


_combine_first = _make_combine(True)
_combine_rest = _make_combine(False)


def _mid_body(c0_ref, x_ref, xh_ref, td_ref, th1_ref, dinv_ref,
              hid_ref, g_ref, out_ref):
    h = jnp.dot(x_ref[...], td_ref[...], preferred_element_type=jnp.float32)
    h += jnp.dot(xh_ref[...], th1_ref[...], preferred_element_type=jnp.float32)
    h = jnp.maximum(h, 0.0)
    hid_ref[...] = h
    g_ref[...] = dinv_ref[...] * h
    out_ref[...] = c0_ref[0, 0] * h


_mid = pl.pallas_call(
    _mid_body,
    grid=(_GRID,),
    in_specs=[
        pl.BlockSpec(memory_space=pltpu.SMEM),
        pl.BlockSpec((_TCR, D), lambda i: (i, 0)),
        pl.BlockSpec((_TCR, D), lambda i: (i, 0)),
        pl.BlockSpec((D, D), lambda i: (0, 0)),
        pl.BlockSpec((D, D), lambda i: (0, 0)),
        pl.BlockSpec((_TCR, D), lambda i: (i, 0)),
    ],
    out_specs=[
        pl.BlockSpec((_TCR, D), lambda i: (i, 0)),
        pl.BlockSpec((_TCR, D), lambda i: (i, 0)),
        pl.BlockSpec((_TCR, D), lambda i: (i, 0)),
    ],
    out_shape=[jax.ShapeDtypeStruct((N, D), jnp.float32)] * 3,
)


def _final_body(h_ref, hh_ref, th_ref, th2_ref, o_ref):
    z = jnp.dot(h_ref[...], th_ref[...], preferred_element_type=jnp.float32)
    z += jnp.dot(hh_ref[...], th2_ref[...], preferred_element_type=jnp.float32)
    m = jnp.max(z, axis=1, keepdims=True)
    lse = m + jnp.log(jnp.sum(jnp.exp(z - m), axis=1, keepdims=True))
    o_ref[...] = z - lse


_final = pl.pallas_call(
    _final_body,
    grid=(_GRID,),
    in_specs=[
        pl.BlockSpec((_TCR, D), lambda i: (i, 0)),
        pl.BlockSpec((_TCR, D), lambda i: (i, 0)),
        pl.BlockSpec((D, D), lambda i: (0, 0)),
        pl.BlockSpec((D, D), lambda i: (0, 0)),
    ],
    out_specs=pl.BlockSpec((_TCR, D), lambda i: (i, 0)),
    out_shape=jax.ShapeDtypeStruct((N, D), jnp.float32),
)


_M30 = np.arange(30, dtype=np.float32)
_LGAMMA = np.array(
    [[math.lgamma(m + 1.0) + math.lgamma(m + k + 1.0) for m in range(30)]
     for k in range(K)], dtype=np.float32)


def _coeffs(t):
    lt = jnp.log(t / 2.0)
    cs = []
    for k in range(K):
        ik = jnp.sum(jnp.exp((2.0 * _M30 + k) * lt - _LGAMMA[k]))
        ck = ik if k == 0 else 2.0 * ((-1.0) ** k) * ik
        cs.append(jnp.reshape(ck.astype(jnp.float32), (1, 1)))
    return cs


def _heat_sweep(g0, out_acc, x0, cr, zeros, dinv, cs):
    g = g0
    tm2 = x0
    tm1 = None
    for k in range(1, K):
        p = _seg(g, cr, zeros).reshape(NC, N, D)
        comb = _combine_first if k == 1 else _combine_rest
        tk, g, out_acc = comb(cs[k], p, tm2, dinv, out_acc)
        tm2, tm1 = (x0, tk) if k == 1 else (tm1, tk)
    return out_acc


def kernel(x, edge_index, theta_direct, theta_heat1, theta_hidden,
           theta_heat2, t):
    row = edge_index[0]
    col = edge_index[1]
    rowp = jnp.concatenate(
        [row.reshape(NW, EPW),
         jnp.full((NW, EPAD), N, jnp.int32)], axis=1).reshape(NW, NCHUNK, 1,
                                                              CH)
    colp = jnp.concatenate(
        [col.reshape(NW, EPW),
         jnp.zeros((NW, EPAD), jnp.int32)], axis=1).reshape(NW, NCHUNK, 1, CH)
    cr = jnp.concatenate([colp, rowp], axis=2)
    zeros = jnp.zeros((CH, D), jnp.float32)
    ones = jnp.ones((N, D), jnp.float32)
    cs = _coeffs(t)

    pdeg = _seg(ones, cr, zeros).reshape(NC, N, D)
    dinv, g0, out1 = _prep(cs[0], pdeg, x)
    x_heat = _heat_sweep(g0, out1, x, cr, zeros, dinv, cs)

    hidden, gh0, out2 = _mid(cs[0], x, x_heat, theta_direct, theta_heat1,
                             dinv)
    hidden_heat = _heat_sweep(gh0, out2, hidden, cr, zeros, dinv, cs)

    return _final(hidden, hidden_heat, theta_hidden, theta_heat2)


# speedup vs baseline: 4.3428x; 2.0075x over previous
import functools
import math

import jax
import jax.numpy as jnp
import numpy as np
from jax import lax
from jax.experimental import pallas as pl
from jax.experimental.pallas import tpu as pltpu
from jax.experimental.pallas import tpu_sc as plsc

N = 10000
E = 320000
D = 128
K = 10

NC = 2
NS = 16
NW = NC * NS
EPW = E // NW
CH = 120
NCHUNK = 84
EPAD = NCHUNK * CH - EPW
NACC = 10240
RPS = NACC // NS

HOP = 64
_TCR = 1000
_GRID = N // _TCR


def _seg_body(v_hbm, cr_hbm, zero_hbm, p_hbm,
              colv0, colv1, colv2, colv3, rowv0, rowv1, rowv2, rowv3,
              gbuf, gbuf1, gbuf2, acc,
              isem0, isem1, isem2, isem3,
              gsem0, gsem1, gsem2, ssem0, ssem1, ssem2):
    c = lax.axis_index("c")
    s = lax.axis_index("s")
    wid = c * NS + s
    colv = (colv0, colv1, colv2, colv3)
    rowv = (rowv0, rowv1, rowv2, rowv3)
    gbufs = (gbuf, gbuf1, gbuf2)
    isems = (isem0, isem1, isem2, isem3)
    gsems = (gsem0, gsem1, gsem2)
    ssems = (ssem0, ssem1, ssem2)

    pltpu.sync_copy(zero_hbm, gbuf)
    rbase = pl.multiple_of(s * RPS, 8)
    for h in range(RPS // HOP):
        pltpu.async_copy(gbuf.at[pl.ds(0, HOP)],
                         acc.at[pl.ds(rbase + h * HOP, HOP)],
                         ssems[h % 3])
    for h in range(RPS // HOP):
        pltpu.make_async_copy(zero_hbm.at[pl.ds(0, HOP)],
                              gbuf.at[pl.ds(0, HOP)], ssems[h % 3]).wait()
    plsc.subcore_barrier()

    def _ifire(j, b):
        pltpu.async_copy(cr_hbm.at[wid, j, 0], colv[b], isems[b])
        pltpu.async_copy(cr_hbm.at[wid, j, 1], rowv[b], isems[b])

    def _iwait(b):
        pltpu.make_async_copy(cr_hbm.at[0, 0, 0], colv[b], isems[b]).wait()
        pltpu.make_async_copy(cr_hbm.at[0, 0, 1], rowv[b], isems[b]).wait()

    def _swait(b):
        pltpu.make_async_copy(zero_hbm, gbufs[b], ssems[b]).wait()

    def _gwait(b):
        pltpu.make_async_copy(zero_hbm, gbufs[b], gsems[b]).wait()

    _ifire(0, 0)
    _ifire(1, 1)
    _iwait(0)
    pltpu.async_copy(v_hbm.at[colv[0]], gbufs[0], gsems[0])

    def block(p, carry):
        for i in range(12):
            j = p * 12 + i

            @pl.when(j + 1 < NCHUNK)
            def _():
                nb = (i + 1) % 3

                @pl.when(j >= 2)
                def _():
                    _swait(nb)
                _iwait((i + 1) % 4)
                pltpu.async_copy(v_hbm.at[colv[(i + 1) % 4]], gbufs[nb],
                                 gsems[nb])

            @pl.when(j + 2 < NCHUNK)
            def _():
                _ifire(j + 2, (i + 2) % 4)

            _gwait(i % 3)
            pltpu.async_copy(gbufs[i % 3], acc.at[rowv[i % 4]], ssems[i % 3],
                             add=True)
        return carry

    lax.fori_loop(0, NCHUNK // 12, block, 0)
    for b in range(3):
        _swait(b)
    plsc.subcore_barrier()

    def _co(h):
        b = h % 3
        if h >= 3:
            pltpu.make_async_copy(zero_hbm.at[pl.ds(0, HOP)],
                                  gbufs[b].at[pl.ds(0, HOP)],
                                  gsems[b]).wait()
        rb = pl.multiple_of(rbase + h * HOP, 8)
        pltpu.sync_copy(acc.at[pl.ds(rb, HOP)], gbufs[b].at[pl.ds(0, HOP)])
        pltpu.async_copy(gbufs[b].at[pl.ds(0, HOP)],
                         p_hbm.at[pl.ds(c * N + rb, HOP)], gsems[b])

    _NHL = (N - (NS - 1) * RPS) // HOP
    for h in range(_NHL):
        _co(h)

    @pl.when(s < NS - 1)
    def _():
        for h in range(_NHL, RPS // HOP):
            _co(h)

    for b in range(3):
        pltpu.make_async_copy(zero_hbm.at[pl.ds(0, HOP)],
                              gbufs[b].at[pl.ds(0, HOP)], gsems[b]).wait()

    _TB = (NS - 1) * RPS + _NHL * HOP
    if N > _TB:
        @pl.when(s == NS - 1)
        def _():
            pltpu.sync_copy(acc.at[pl.ds(_TB, N - _TB)],
                            gbuf.at[pl.ds(0, N - _TB)])
            pltpu.sync_copy(gbuf.at[pl.ds(0, N - _TB)],
                            p_hbm.at[pl.ds(c * N + _TB, N - _TB)])


_seg = pl.kernel(
    _seg_body,
    out_type=jax.ShapeDtypeStruct((NC * N, D), jnp.float32),
    mesh=plsc.VectorSubcoreMesh(core_axis_name="c", subcore_axis_name="s"),
    scratch_types=(
        [pltpu.VMEM((CH,), jnp.int32)] * 4
        + [pltpu.VMEM((CH,), jnp.int32)] * 4
        + [pltpu.VMEM((CH, D), jnp.float32)] * 3
        + [pltpu.VMEM_SHARED((NACC, D), jnp.float32)]
        + [pltpu.SemaphoreType.DMA] * 10
    ),
)


def _prep_body(c0_ref, p_ref, x_ref, dinv_ref, g_ref, out_ref):
    s = p_ref[0] + p_ref[1]
    dinv = jnp.where(s > 0, lax.rsqrt(jnp.maximum(s, 1e-12)), 0.0)
    x = x_ref[...]
    dinv_ref[...] = dinv
    g_ref[...] = dinv * x
    out_ref[...] = c0_ref[0, 0] * x


_prep = pl.pallas_call(
    _prep_body,
    grid=(_GRID,),
    in_specs=[
        pl.BlockSpec(memory_space=pltpu.SMEM),
        pl.BlockSpec((2, _TCR, D), lambda i: (0, i, 0)),
        pl.BlockSpec((_TCR, D), lambda i: (i, 0)),
    ],
    out_specs=[
        pl.BlockSpec((_TCR, D), lambda i: (i, 0)),
        pl.BlockSpec((_TCR, D), lambda i: (i, 0)),
        pl.BlockSpec((_TCR, D), lambda i: (i, 0)),
    ],
    out_shape=[jax.ShapeDtypeStruct((N, D), jnp.float32)] * 3,
)


def _combine_body(ck_ref, p_ref, tm2_ref, dinv_ref, outin_ref,
                  t_ref, g_ref, outnew_ref, *, first):
    s = p_ref[0] + p_ref[1]
    dinv = dinv_ref[...]
    if first:
        t = -dinv * s
    else:
        t = -2.0 * (dinv * s) - tm2_ref[...]
    t_ref[...] = t
    g_ref[...] = dinv * t
    outnew_ref[...] = outin_ref[...] + ck_ref[0, 0] * t


def _make_combine(first):
    return pl.pallas_call(
        functools.partial(_combine_body, first=first),
        grid=(_GRID,),
        in_specs=[
            pl.BlockSpec(memory_space=pltpu.SMEM),
            pl.BlockSpec((2, _TCR, D), lambda i: (0, i, 0)),
            pl.BlockSpec((_TCR, D), lambda i: (i, 0)),
            pl.BlockSpec((_TCR, D), lambda i: (i, 0)),
            pl.BlockSpec((_TCR, D), lambda i: (i, 0)),
        ],
        out_specs=[
            pl.BlockSpec((_TCR, D), lambda i: (i, 0)),
            pl.BlockSpec((_TCR, D), lambda i: (i, 0)),
            pl.BlockSpec((_TCR, D), lambda i: (i, 0)),
        ],
        out_shape=[jax.ShapeDtypeStruct((N, D), jnp.float32)] * 3,
    )


_combine_first = _make_combine(True)
_combine_rest = _make_combine(False)


def _mid_body(c0_ref, x_ref, xh_ref, td_ref, th1_ref, dinv_ref,
              hid_ref, g_ref, out_ref):
    h = jnp.dot(x_ref[...], td_ref[...], preferred_element_type=jnp.float32)
    h += jnp.dot(xh_ref[...], th1_ref[...], preferred_element_type=jnp.float32)
    h = jnp.maximum(h, 0.0)
    hid_ref[...] = h
    g_ref[...] = dinv_ref[...] * h
    out_ref[...] = c0_ref[0, 0] * h


_mid = pl.pallas_call(
    _mid_body,
    grid=(_GRID,),
    in_specs=[
        pl.BlockSpec(memory_space=pltpu.SMEM),
        pl.BlockSpec((_TCR, D), lambda i: (i, 0)),
        pl.BlockSpec((_TCR, D), lambda i: (i, 0)),
        pl.BlockSpec((D, D), lambda i: (0, 0)),
        pl.BlockSpec((D, D), lambda i: (0, 0)),
        pl.BlockSpec((_TCR, D), lambda i: (i, 0)),
    ],
    out_specs=[
        pl.BlockSpec((_TCR, D), lambda i: (i, 0)),
        pl.BlockSpec((_TCR, D), lambda i: (i, 0)),
        pl.BlockSpec((_TCR, D), lambda i: (i, 0)),
    ],
    out_shape=[jax.ShapeDtypeStruct((N, D), jnp.float32)] * 3,
)


def _final_body(h_ref, hh_ref, th_ref, th2_ref, o_ref):
    z = jnp.dot(h_ref[...], th_ref[...], preferred_element_type=jnp.float32)
    z += jnp.dot(hh_ref[...], th2_ref[...], preferred_element_type=jnp.float32)
    m = jnp.max(z, axis=1, keepdims=True)
    lse = m + jnp.log(jnp.sum(jnp.exp(z - m), axis=1, keepdims=True))
    o_ref[...] = z - lse


_final = pl.pallas_call(
    _final_body,
    grid=(_GRID,),
    in_specs=[
        pl.BlockSpec((_TCR, D), lambda i: (i, 0)),
        pl.BlockSpec((_TCR, D), lambda i: (i, 0)),
        pl.BlockSpec((D, D), lambda i: (0, 0)),
        pl.BlockSpec((D, D), lambda i: (0, 0)),
    ],
    out_specs=pl.BlockSpec((_TCR, D), lambda i: (i, 0)),
    out_shape=jax.ShapeDtypeStruct((N, D), jnp.float32),
)


_M30 = np.arange(30, dtype=np.float32)
_LGAMMA = np.array(
    [[math.lgamma(m + 1.0) + math.lgamma(m + k + 1.0) for m in range(30)]
     for k in range(K)], dtype=np.float32)


def _coeffs(t):
    lt = jnp.log(t / 2.0)
    cs = []
    for k in range(K):
        ik = jnp.sum(jnp.exp((2.0 * _M30 + k) * lt - _LGAMMA[k]))
        ck = ik if k == 0 else 2.0 * ((-1.0) ** k) * ik
        cs.append(jnp.reshape(ck.astype(jnp.float32), (1, 1)))
    return cs


def _heat_sweep(g0, out_acc, x0, cr, zeros, dinv, cs, need):
    p = _seg(g0, cr, zeros).reshape(NC, N, D)
    t1, g, out_acc = _combine_first(cs[1], p, x0, dinv, out_acc)
    carry = (t1, x0, g, out_acc)

    for k in range(2, K):
        def _step(c, _k=k):
            tm1, tm2, g_in, out_in = c
            p = _seg(g_in, cr, zeros).reshape(NC, N, D)
            tk, gk, outk = _combine_rest(cs[_k], p, tm2, dinv, out_in)
            return (tk, tm1, gk, outk)

        carry = lax.cond(need[k], _step, lambda c: c, carry)
    return carry[3]


def kernel(x, edge_index, theta_direct, theta_heat1, theta_hidden,
           theta_heat2, t):
    row = edge_index[0]
    col = edge_index[1]
    rowp = jnp.concatenate(
        [row.reshape(NW, EPW),
         jnp.full((NW, EPAD), N, jnp.int32)], axis=1).reshape(NW, NCHUNK, 1,
                                                              CH)
    colp = jnp.concatenate(
        [col.reshape(NW, EPW),
         jnp.zeros((NW, EPAD), jnp.int32)], axis=1).reshape(NW, NCHUNK, 1, CH)
    cr = jnp.concatenate([colp, rowp], axis=2)
    zeros = jnp.zeros((CH, D), jnp.float32)
    ones = jnp.ones((N, D), jnp.float32)
    cs = _coeffs(t)

    absc = [jnp.abs(c[0, 0]) for c in cs]
    sufmax = list(absc)
    for k in range(K - 2, -1, -1):
        sufmax[k] = jnp.maximum(sufmax[k], sufmax[k + 1])
    need = [sm > 1e-7 * absc[0] for sm in sufmax]

    pdeg = _seg(ones, cr, zeros).reshape(NC, N, D)
    dinv, g0, out1 = _prep(cs[0], pdeg, x)
    x_heat = _heat_sweep(g0, out1, x, cr, zeros, dinv, cs, need)

    hidden, gh0, out2 = _mid(cs[0], x, x_heat, theta_direct, theta_heat1,
                             dinv)
    hidden_heat = _heat_sweep(gh0, out2, hidden, cr, zeros, dinv, cs, need)

    return _final(hidden, hidden_heat, theta_hidden, theta_heat2)
